# baseline (speedup 1.0000x reference)
import functools

import jax
import jax.numpy as jnp
from jax import lax
from jax.experimental import pallas as pl
from jax.experimental.pallas import tpu as pltpu
from jax.experimental.pallas import tpu_sc as plsc

VOCAB = 1000000
B = 16384
OP_DIM = 32
STATS_IN = 4
STATS_H = 16
PRED_DIM = 8
OUT_DIM = 64

_NC = 2
_NS = 16
_NW = _NC * _NS
_BPW = B // _NW
_LSLICE = 128
_NBUF = 16


def _sc_gather_t(table_t, idx, n):
    mesh = plsc.VectorSubcoreMesh(core_axis_name="c", subcore_axis_name="s")
    lanes_mask = _LSLICE - 1
    bpw = n // _NW

    @functools.partial(
        pl.kernel,
        mesh=mesh,
        out_type=jax.ShapeDtypeStruct((OP_DIM, n), jnp.float32),
        scratch_types=[
            pltpu.VMEM((bpw,), jnp.int32),
            pltpu.VMEM((_NBUF, OP_DIM, _LSLICE), jnp.float32),
            pltpu.VMEM((OP_DIM, bpw), jnp.float32),
            pltpu.SemaphoreType.DMA,
        ],
        compiler_params=pltpu.CompilerParams(use_tc_tiling_on_sc=True,
                                             needs_layout_passes=False),
    )
    def k(table_hbm, idx_hbm, out_hbm, idx_v, ring_v, stage_v, sem):
        wid = lax.axis_index("s") * _NC + lax.axis_index("c")
        base = wid * bpw
        pltpu.sync_copy(idx_hbm.at[pl.ds(base, bpw)], idx_v)

        lane_iota = lax.iota(jnp.int32, 16)
        d_lo = lane_iota
        d_hi = lane_iota + 16
        neg_inf = jnp.full((16,), jnp.int32(-2147483648))

        def lane_scalar(v, t):
            return jnp.max(jnp.where(lane_iota == t, v, neg_inf))

        def fire(i_scalar, slot):
            lane0 = pl.multiple_of(i_scalar & ~lanes_mask, _LSLICE)
            pltpu.async_copy(
                table_hbm.at[:, pl.ds(lane0, _LSLICE)],
                ring_v.at[slot],
                sem,
            )

        def wait_slot(slot):
            pltpu.make_async_copy(
                table_hbm.at[:, pl.ds(0, _LSLICE)],
                ring_v.at[slot],
                sem,
            ).wait()

        def extract(i_scalar, j, slot):
            l = i_scalar & lanes_mask
            l_vec = jnp.full((16,), l, jnp.int32)
            j_vec = jnp.full((16,), j, jnp.int32)
            buf = ring_v.at[slot]
            v0 = plsc.load_gather(buf, [d_lo, l_vec])
            v1 = plsc.load_gather(buf, [d_hi, l_vec])
            plsc.store_scatter(stage_v, [d_lo, j_vec], v0)
            plsc.store_scatter(stage_v, [d_hi, j_vec], v1)

        n_grp = bpw // _NBUF

        v0g = idx_v[pl.ds(0, _NBUF)]
        for t in range(_NBUF):
            fire(lane_scalar(v0g, t), t)

        def body(g, v_cur):
            v_next = idx_v[pl.ds((g + 1) * _NBUF, _NBUF)]
            for t in range(_NBUF):
                j = g * _NBUF + t
                wait_slot(t)
                extract(lane_scalar(v_cur, t), j, t)
                fire(lane_scalar(v_next, t), t)
            return v_next

        v_last = lax.fori_loop(0, n_grp - 1, body, v0g)
        for t in range(_NBUF):
            j = (n_grp - 1) * _NBUF + t
            wait_slot(t)
            extract(lane_scalar(v_last, t), j, t)

        pltpu.sync_copy(stage_v, out_hbm.at[:, pl.ds(base, bpw)])

    return k(table_t, idx)


def _dot_t(a, b):
    return lax.dot_general(a, b, dimension_numbers=(((0,), (0,)), ((), ())),
                           preferred_element_type=jnp.float32)


def _tc_body(g_ref, s_ref, p_ref, W1_ref, b1_ref, W2_ref, b2_ref,
             Wout_ref, bout_ref, o_ref):
    s_t = jnp.log1p(s_ref[...])
    h_t = jnp.maximum(_dot_t(W1_ref[...], s_t) + b1_ref[...], 0.0)
    sv_t = _dot_t(W2_ref[...], h_t) + b2_ref[...]
    out_t = (_dot_t(Wout_ref[0:OP_DIM, :], g_ref[...])
             + _dot_t(Wout_ref[OP_DIM:OP_DIM + STATS_H, :], sv_t)
             + _dot_t(Wout_ref[OP_DIM + STATS_H:, :], p_ref[...])
             + bout_ref[...])
    o_ref[...] = out_t


def _tc_dense(g_t, stats_t, pred_t, W1, b1, W2, b2, Wout, bout,
              off_blk, n_blk, prev=None):
    BLK = 2048
    col_spec = lambda d: pl.BlockSpec((d, BLK), lambda i: (0, i + off_blk))
    full = lambda a: pl.BlockSpec(a.shape, lambda i: tuple(0 for _ in a.shape))
    b1_2d = b1.reshape(STATS_H, 1)
    b2_2d = b2.reshape(STATS_H, 1)
    bout_2d = bout.reshape(OUT_DIM, 1)
    g_spec = pl.BlockSpec((OP_DIM, BLK), lambda i: (0, i))
    in_specs = [
        g_spec,
        col_spec(STATS_IN),
        col_spec(PRED_DIM),
        full(W1), full(b1_2d), full(W2), full(b2_2d),
        full(Wout), full(bout_2d),
    ]
    args = [g_t, stats_t, pred_t, W1, b1_2d, W2, b2_2d, Wout, bout_2d]
    body = _tc_body
    aliases = {}
    if prev is not None:
        in_specs.append(pl.BlockSpec(memory_space=pl.ANY))
        args.append(prev)
        aliases = {9: 0}
        body = lambda *refs: _tc_body(*refs[:9], refs[10])
    return pl.pallas_call(
        body,
        grid=(n_blk,),
        in_specs=in_specs,
        out_specs=pl.BlockSpec((OUT_DIM, BLK), lambda i: (0, i + off_blk)),
        out_shape=jax.ShapeDtypeStruct((OUT_DIM, B), jnp.float32),
        input_output_aliases=aliases,
    )(*args)


def kernel(op_idx, stats, pred_feat, emb_table, W1, b1, W2, b2, Wout, bout):
    idx = op_idx.astype(jnp.int32)
    table_t = emb_table.T
    stats_t, pred_t = stats.T, pred_feat.T
    half = B // 2
    g1 = _sc_gather_t(table_t, idx[:half], half)
    g2 = _sc_gather_t(table_t, idx[half:], half)
    w = (W1, b1, W2, b2, Wout, bout)
    o1 = _tc_dense(g1, stats_t, pred_t, *w, off_blk=0, n_blk=half // 2048)
    out_t = _tc_dense(g2, stats_t, pred_t, *w, off_blk=half // 2048,
                      n_blk=half // 2048, prev=o1)
    return out_t.T


# speedup vs baseline: 1.0284x; 1.0284x over previous
import functools

import jax
import jax.numpy as jnp
from jax import lax
from jax.experimental import pallas as pl
from jax.experimental.pallas import tpu as pltpu
from jax.experimental.pallas import tpu_sc as plsc

VOCAB = 1000000
B = 16384
OP_DIM = 32
STATS_IN = 4
STATS_H = 16
PRED_DIM = 8
OUT_DIM = 64

_NC = 2
_NS = 16
_NW = _NC * _NS
_BPW = B // _NW
_LSLICE = 128
_NBUF = 16


def _sc_gather_t(table_t, idx, n):
    mesh = plsc.VectorSubcoreMesh(core_axis_name="c", subcore_axis_name="s")
    lanes_mask = _LSLICE - 1
    bpw = n // _NW

    @functools.partial(
        pl.kernel,
        mesh=mesh,
        out_type=jax.ShapeDtypeStruct((OP_DIM, n), jnp.float32),
        scratch_types=[
            pltpu.VMEM((bpw,), jnp.int32),
            pltpu.VMEM((_NBUF, OP_DIM, _LSLICE), jnp.float32),
            pltpu.VMEM((OP_DIM, bpw), jnp.float32),
            pltpu.SemaphoreType.DMA,
        ],
        compiler_params=pltpu.CompilerParams(use_tc_tiling_on_sc=True,
                                             needs_layout_passes=False),
    )
    def k(table_hbm, idx_hbm, out_hbm, idx_v, ring_v, stage_v, sem):
        wid = lax.axis_index("s") * _NC + lax.axis_index("c")
        base = wid * bpw
        pltpu.sync_copy(idx_hbm.at[pl.ds(base, bpw)], idx_v)

        lane_iota = lax.iota(jnp.int32, 16)
        d_lo = lane_iota
        d_hi = lane_iota + 16
        neg_inf = jnp.full((16,), jnp.int32(-2147483648))

        def lane_scalar(v, t):
            return jnp.max(jnp.where(lane_iota == t, v, neg_inf))

        def fire(i_scalar, slot):
            lane0 = pl.multiple_of(i_scalar & ~lanes_mask, _LSLICE)
            pltpu.async_copy(
                table_hbm.at[:, pl.ds(lane0, _LSLICE)],
                ring_v.at[slot],
                sem,
            )

        def wait_slot(slot):
            pltpu.make_async_copy(
                table_hbm.at[:, pl.ds(0, _LSLICE)],
                ring_v.at[slot],
                sem,
            ).wait()

        def extract(i_scalar, j, slot):
            l = i_scalar & lanes_mask
            l_vec = jnp.full((16,), l, jnp.int32)
            j_vec = jnp.full((16,), j, jnp.int32)
            buf = ring_v.at[slot]
            v0 = plsc.load_gather(buf, [d_lo, l_vec])
            v1 = plsc.load_gather(buf, [d_hi, l_vec])
            plsc.store_scatter(stage_v, [d_lo, j_vec], v0)
            plsc.store_scatter(stage_v, [d_hi, j_vec], v1)

        n_grp = bpw // _NBUF

        v0g = idx_v[pl.ds(0, _NBUF)]
        for t in range(_NBUF):
            fire(lane_scalar(v0g, t), t)

        def body(g, v_cur):
            v_next = idx_v[pl.ds((g + 1) * _NBUF, _NBUF)]
            for t in range(_NBUF):
                j = g * _NBUF + t
                wait_slot(t)
                extract(lane_scalar(v_cur, t), j, t)
                fire(lane_scalar(v_next, t), t)
            return v_next

        v_last = lax.fori_loop(0, n_grp - 1, body, v0g)
        for t in range(_NBUF):
            j = (n_grp - 1) * _NBUF + t
            wait_slot(t)
            extract(lane_scalar(v_last, t), j, t)

        pltpu.sync_copy(stage_v, out_hbm.at[:, pl.ds(base, bpw)])

    return k(table_t, idx)


def _dot_t(a, b):
    return lax.dot_general(a, b, dimension_numbers=(((0,), (0,)), ((), ())),
                           preferred_element_type=jnp.float32)


def _tc_body(g_ref, s_ref, p_ref, W1_ref, b1_ref, W2_ref, b2_ref,
             Wout_ref, bout_ref, o_ref):
    s_t = jnp.log1p(s_ref[...])
    h_t = jnp.maximum(_dot_t(W1_ref[...], s_t) + b1_ref[...], 0.0)
    sv_t = _dot_t(W2_ref[...], h_t) + b2_ref[...]
    out_t = (_dot_t(Wout_ref[0:OP_DIM, :], g_ref[...])
             + _dot_t(Wout_ref[OP_DIM:OP_DIM + STATS_H, :], sv_t)
             + _dot_t(Wout_ref[OP_DIM + STATS_H:, :], p_ref[...])
             + bout_ref[...])
    o_ref[...] = out_t


def _tc_dense(g_t, stats_t, pred_t, W1, b1, W2, b2, Wout, bout,
              off_blk, n_blk, prev=None):
    BLK = 2048
    col_spec = lambda d: pl.BlockSpec((d, BLK), lambda i: (0, i + off_blk))
    full = lambda a: pl.BlockSpec(a.shape, lambda i: tuple(0 for _ in a.shape))
    b1_2d = b1.reshape(STATS_H, 1)
    b2_2d = b2.reshape(STATS_H, 1)
    bout_2d = bout.reshape(OUT_DIM, 1)
    g_spec = pl.BlockSpec((OP_DIM, BLK), lambda i: (0, i))
    in_specs = [
        g_spec,
        col_spec(STATS_IN),
        col_spec(PRED_DIM),
        full(W1), full(b1_2d), full(W2), full(b2_2d),
        full(Wout), full(bout_2d),
    ]
    args = [g_t, stats_t, pred_t, W1, b1_2d, W2, b2_2d, Wout, bout_2d]
    body = _tc_body
    aliases = {}
    if prev is not None:
        in_specs.append(pl.BlockSpec(memory_space=pl.ANY))
        args.append(prev)
        aliases = {9: 0}
        body = lambda *refs: _tc_body(*refs[:9], refs[10])
    return pl.pallas_call(
        body,
        grid=(n_blk,),
        in_specs=in_specs,
        out_specs=pl.BlockSpec((OUT_DIM, BLK), lambda i: (0, i + off_blk)),
        out_shape=jax.ShapeDtypeStruct((OUT_DIM, B), jnp.float32),
        input_output_aliases=aliases,
    )(*args)


def kernel(op_idx, stats, pred_feat, emb_table, W1, b1, W2, b2, Wout, bout):
    idx = op_idx.astype(jnp.int32)
    g_t = _sc_gather_t(emb_table.T, idx, B)
    out_t = _tc_dense(g_t, stats.T, pred_feat.T, W1, b1, W2, b2, Wout, bout,
                      off_blk=0, n_blk=B // 2048)
    return out_t.T


# speedup vs baseline: 1.0487x; 1.0197x over previous
import functools

import jax
import jax.numpy as jnp
from jax import lax
from jax.experimental import pallas as pl
from jax.experimental.pallas import tpu as pltpu
from jax.experimental.pallas import tpu_sc as plsc

VOCAB = 1000000
B = 16384
OP_DIM = 32
STATS_IN = 4
STATS_H = 16
PRED_DIM = 8
OUT_DIM = 64

_NC = 2
_NS = 16
_NW = _NC * _NS
_BPW = B // _NW
_LSLICE = 128
_NBUF = 16


def _sc_gather_t(table_t, idx, n):
    mesh = plsc.VectorSubcoreMesh(core_axis_name="c", subcore_axis_name="s")
    lanes_mask = _LSLICE - 1
    bpw = n // _NW

    @functools.partial(
        pl.kernel,
        mesh=mesh,
        out_type=jax.ShapeDtypeStruct((OP_DIM, n), jnp.float32),
        scratch_types=[
            pltpu.VMEM((bpw,), jnp.int32),
            pltpu.VMEM((_NBUF, OP_DIM, _LSLICE), jnp.float32),
            pltpu.VMEM((OP_DIM, bpw), jnp.float32),
            pltpu.SemaphoreType.DMA,
        ],
        compiler_params=pltpu.CompilerParams(use_tc_tiling_on_sc=True,
                                             needs_layout_passes=False),
    )
    def k(table_hbm, idx_hbm, out_hbm, idx_v, ring_v, stage_v, sem):
        wid = lax.axis_index("s") * _NC + lax.axis_index("c")
        base = wid * bpw
        pltpu.sync_copy(idx_hbm.at[pl.ds(base, bpw)], idx_v)

        lane_iota = lax.iota(jnp.int32, 16)
        d_lo = lane_iota
        d_hi = lane_iota + 16
        neg_inf = jnp.full((16,), jnp.int32(-2147483648))

        def lane_scalar(v, t):
            return jnp.max(jnp.where(lane_iota == t, v, neg_inf))

        def fire(i_scalar, slot):
            lane0 = pl.multiple_of(i_scalar & ~lanes_mask, _LSLICE)
            pltpu.async_copy(
                table_hbm.at[:, pl.ds(lane0, _LSLICE)],
                ring_v.at[slot],
                sem,
            )

        def wait_slot(slot):
            pltpu.make_async_copy(
                table_hbm.at[:, pl.ds(0, _LSLICE)],
                ring_v.at[slot],
                sem,
            ).wait()

        def extract(i_scalar, j, slot):
            l = i_scalar & lanes_mask
            l_vec = jnp.full((16,), l, jnp.int32)
            j_vec = jnp.full((16,), j, jnp.int32)
            buf = ring_v.at[slot]
            v0 = plsc.load_gather(buf, [d_lo, l_vec])
            v1 = plsc.load_gather(buf, [d_hi, l_vec])
            plsc.store_scatter(stage_v, [d_lo, j_vec], v0)
            plsc.store_scatter(stage_v, [d_hi, j_vec], v1)

        n_grp = bpw // _NBUF

        v0g = idx_v[pl.ds(0, _NBUF)]
        for t in range(_NBUF):
            fire(lane_scalar(v0g, t), t)

        def body(g, v_cur):
            v_next = idx_v[pl.ds((g + 1) * _NBUF, _NBUF)]
            for t in range(_NBUF):
                j = g * _NBUF + t
                wait_slot(t)
                extract(lane_scalar(v_cur, t), j, t)
                fire(lane_scalar(v_next, t), t)
            return v_next

        v_last = lax.fori_loop(0, n_grp - 1, body, v0g)
        for t in range(_NBUF):
            j = (n_grp - 1) * _NBUF + t
            wait_slot(t)
            extract(lane_scalar(v_last, t), j, t)

        pltpu.sync_copy(stage_v, out_hbm.at[:, pl.ds(base, bpw)])

    return k(table_t, idx)


def _dot_t(a, b):
    return lax.dot_general(a, b, dimension_numbers=(((0,), (0,)), ((), ())),
                           preferred_element_type=jnp.float32)


def _tc_body(g_ref, s_ref, p_ref, W1_ref, b1_ref, W2_ref, b2_ref,
             Wout_ref, bout_ref, o_ref):
    s_t = jnp.log1p(s_ref[...])
    h_t = jnp.maximum(_dot_t(W1_ref[...], s_t) + b1_ref[...], 0.0)
    sv_t = _dot_t(W2_ref[...], h_t) + b2_ref[...]
    out_t = (_dot_t(Wout_ref[0:OP_DIM, :], g_ref[...])
             + _dot_t(Wout_ref[OP_DIM:OP_DIM + STATS_H, :], sv_t)
             + _dot_t(Wout_ref[OP_DIM + STATS_H:, :], p_ref[...])
             + bout_ref[...])
    o_ref[...] = out_t


def _tc_dense(g_t, stats_t, pred_t, W1, b1, W2, b2, Wout, bout,
              off_blk, n_blk, prev=None):
    BLK = 4096
    col_spec = lambda d: pl.BlockSpec((d, BLK), lambda i: (0, i + off_blk))
    full = lambda a: pl.BlockSpec(a.shape, lambda i: tuple(0 for _ in a.shape))
    b1_2d = b1.reshape(STATS_H, 1)
    b2_2d = b2.reshape(STATS_H, 1)
    bout_2d = bout.reshape(OUT_DIM, 1)
    g_spec = pl.BlockSpec((OP_DIM, BLK), lambda i: (0, i))
    in_specs = [
        g_spec,
        col_spec(STATS_IN),
        col_spec(PRED_DIM),
        full(W1), full(b1_2d), full(W2), full(b2_2d),
        full(Wout), full(bout_2d),
    ]
    args = [g_t, stats_t, pred_t, W1, b1_2d, W2, b2_2d, Wout, bout_2d]
    body = _tc_body
    aliases = {}
    if prev is not None:
        in_specs.append(pl.BlockSpec(memory_space=pl.ANY))
        args.append(prev)
        aliases = {9: 0}
        body = lambda *refs: _tc_body(*refs[:9], refs[10])
    return pl.pallas_call(
        body,
        grid=(n_blk,),
        in_specs=in_specs,
        out_specs=pl.BlockSpec((OUT_DIM, BLK), lambda i: (0, i + off_blk)),
        out_shape=jax.ShapeDtypeStruct((OUT_DIM, B), jnp.float32),
        input_output_aliases=aliases,
    )(*args)


def kernel(op_idx, stats, pred_feat, emb_table, W1, b1, W2, b2, Wout, bout):
    idx = op_idx.astype(jnp.int32)
    g_t = _sc_gather_t(emb_table.T, idx, B)
    out_t = _tc_dense(g_t, stats.T, pred_feat.T, W1, b1, W2, b2, Wout, bout,
                      off_blk=0, n_blk=B // 4096)
    return out_t.T


# speedup vs baseline: 1.0586x; 1.0094x over previous
import functools

import jax
import jax.numpy as jnp
from jax import lax
from jax.experimental import pallas as pl
from jax.experimental.pallas import tpu as pltpu
from jax.experimental.pallas import tpu_sc as plsc

VOCAB = 1000000
B = 16384
OP_DIM = 32
STATS_IN = 4
STATS_H = 16
PRED_DIM = 8
OUT_DIM = 64

_NC = 2
_NS = 16
_NW = _NC * _NS
_BPW = B // _NW
_LSLICE = 128
_NBUF = 16


def _sc_gather_t(table_t, idx, n):
    mesh = plsc.VectorSubcoreMesh(core_axis_name="c", subcore_axis_name="s")
    lanes_mask = _LSLICE - 1
    bpw = n // _NW

    @functools.partial(
        pl.kernel,
        mesh=mesh,
        out_type=jax.ShapeDtypeStruct((OP_DIM, n), jnp.float32),
        scratch_types=[
            pltpu.VMEM((bpw,), jnp.int32),
            pltpu.VMEM((_NBUF, OP_DIM, _LSLICE), jnp.float32),
            pltpu.VMEM((OP_DIM, bpw), jnp.float32),
            pltpu.SemaphoreType.DMA,
        ],
        compiler_params=pltpu.CompilerParams(use_tc_tiling_on_sc=True,
                                             needs_layout_passes=False),
    )
    def k(table_hbm, idx_hbm, out_hbm, idx_v, ring_v, stage_v, sem):
        wid = lax.axis_index("s") * _NC + lax.axis_index("c")
        base = wid * bpw
        pltpu.sync_copy(idx_hbm.at[pl.ds(base, bpw)], idx_v)

        lane_iota = lax.iota(jnp.int32, 16)
        d_lo = lane_iota
        d_hi = lane_iota + 16
        neg_inf = jnp.full((16,), jnp.int32(-2147483648))

        def lane_scalar(v, t):
            return jnp.max(jnp.where(lane_iota == t, v, neg_inf))

        def fire(i_scalar, slot):
            lane0 = pl.multiple_of(i_scalar & ~lanes_mask, _LSLICE)
            pltpu.async_copy(
                table_hbm.at[:, pl.ds(lane0, _LSLICE)],
                ring_v.at[slot],
                sem,
            )

        def wait_slot(slot):
            pltpu.make_async_copy(
                table_hbm.at[:, pl.ds(0, _LSLICE)],
                ring_v.at[slot],
                sem,
            ).wait()

        def extract(i_scalar, j, slot):
            l = i_scalar & lanes_mask
            l_vec = jnp.full((16,), l, jnp.int32)
            j_vec = jnp.full((16,), j, jnp.int32)
            buf = ring_v.at[slot]
            v0 = plsc.load_gather(buf, [d_lo, l_vec])
            v1 = plsc.load_gather(buf, [d_hi, l_vec])
            plsc.store_scatter(stage_v, [d_lo, j_vec], v0)
            plsc.store_scatter(stage_v, [d_hi, j_vec], v1)

        n_grp = bpw // _NBUF

        v0g = idx_v[pl.ds(0, _NBUF)]
        for t in range(_NBUF):
            fire(lane_scalar(v0g, t), t)

        def body(g, v_cur):
            v_next = idx_v[pl.ds((g + 1) * _NBUF, _NBUF)]
            for t in range(_NBUF):
                j = g * _NBUF + t
                wait_slot(t)
                extract(lane_scalar(v_cur, t), j, t)
                fire(lane_scalar(v_next, t), t)
            return v_next

        v_last = lax.fori_loop(0, n_grp - 1, body, v0g)
        for t in range(_NBUF):
            j = (n_grp - 1) * _NBUF + t
            wait_slot(t)
            extract(lane_scalar(v_last, t), j, t)

        pltpu.sync_copy(stage_v, out_hbm.at[:, pl.ds(base, bpw)])

    return k(table_t, idx)


def _dot_t(a, b):
    return lax.dot_general(a, b, dimension_numbers=(((0,), (0,)), ((), ())),
                           preferred_element_type=jnp.float32)


def _tc_body(g_ref, s_ref, p_ref, W1_ref, b1_ref, W2_ref, b2_ref,
             Wout_ref, bout_ref, o_ref):
    s_t = jnp.log1p(s_ref[...])
    h_t = jnp.maximum(_dot_t(W1_ref[...], s_t) + b1_ref[...], 0.0)
    sv_t = _dot_t(W2_ref[...], h_t) + b2_ref[...]
    out_t = (_dot_t(Wout_ref[0:OP_DIM, :], g_ref[...])
             + _dot_t(Wout_ref[OP_DIM:OP_DIM + STATS_H, :], sv_t)
             + _dot_t(Wout_ref[OP_DIM + STATS_H:, :], p_ref[...])
             + bout_ref[...])
    o_ref[...] = out_t


def _tc_dense(g_t, stats_t, pred_t, W1, b1, W2, b2, Wout, bout,
              off_blk, n_blk, prev=None):
    BLK = 8192
    col_spec = lambda d: pl.BlockSpec((d, BLK), lambda i: (0, i + off_blk))
    full = lambda a: pl.BlockSpec(a.shape, lambda i: tuple(0 for _ in a.shape))
    b1_2d = b1.reshape(STATS_H, 1)
    b2_2d = b2.reshape(STATS_H, 1)
    bout_2d = bout.reshape(OUT_DIM, 1)
    g_spec = pl.BlockSpec((OP_DIM, BLK), lambda i: (0, i))
    in_specs = [
        g_spec,
        col_spec(STATS_IN),
        col_spec(PRED_DIM),
        full(W1), full(b1_2d), full(W2), full(b2_2d),
        full(Wout), full(bout_2d),
    ]
    args = [g_t, stats_t, pred_t, W1, b1_2d, W2, b2_2d, Wout, bout_2d]
    body = _tc_body
    aliases = {}
    if prev is not None:
        in_specs.append(pl.BlockSpec(memory_space=pl.ANY))
        args.append(prev)
        aliases = {9: 0}
        body = lambda *refs: _tc_body(*refs[:9], refs[10])
    return pl.pallas_call(
        body,
        grid=(n_blk,),
        in_specs=in_specs,
        out_specs=pl.BlockSpec((OUT_DIM, BLK), lambda i: (0, i + off_blk)),
        out_shape=jax.ShapeDtypeStruct((OUT_DIM, B), jnp.float32),
        input_output_aliases=aliases,
    )(*args)


def kernel(op_idx, stats, pred_feat, emb_table, W1, b1, W2, b2, Wout, bout):
    idx = op_idx.astype(jnp.int32)
    g_t = _sc_gather_t(emb_table.T, idx, B)
    out_t = _tc_dense(g_t, stats.T, pred_feat.T, W1, b1, W2, b2, Wout, bout,
                      off_blk=0, n_blk=B // 8192)
    return out_t.T


# speedup vs baseline: 1.0586x; 1.0000x over previous
import functools

import jax
import jax.numpy as jnp
from jax import lax
from jax.experimental import pallas as pl
from jax.experimental.pallas import tpu as pltpu
from jax.experimental.pallas import tpu_sc as plsc

VOCAB = 1000000
B = 16384
OP_DIM = 32
STATS_IN = 4
STATS_H = 16
PRED_DIM = 8
OUT_DIM = 64

_NC = 2
_NS = 16
_NW = _NC * _NS
_BPW = B // _NW
_LSLICE = 128
_NBUF = 16


def _sc_gather_t(table_t, idx, n):
    mesh = plsc.VectorSubcoreMesh(core_axis_name="c", subcore_axis_name="s")
    lanes_mask = _LSLICE - 1
    bpw = n // _NW

    @functools.partial(
        pl.kernel,
        mesh=mesh,
        out_type=jax.ShapeDtypeStruct((OP_DIM, n), jnp.float32),
        scratch_types=[
            pltpu.VMEM((bpw,), jnp.int32),
            pltpu.VMEM((_NBUF, OP_DIM, _LSLICE), jnp.float32),
            pltpu.VMEM((OP_DIM, bpw), jnp.float32),
            pltpu.SemaphoreType.DMA,
        ],
        compiler_params=pltpu.CompilerParams(use_tc_tiling_on_sc=True,
                                             needs_layout_passes=False),
    )
    def k(table_hbm, idx_hbm, out_hbm, idx_v, ring_v, stage_v, sem):
        wid = lax.axis_index("s") * _NC + lax.axis_index("c")
        base = wid * bpw
        pltpu.sync_copy(idx_hbm.at[pl.ds(base, bpw)], idx_v)

        lane_iota = lax.iota(jnp.int32, 16)
        d_lo = lane_iota
        d_hi = lane_iota + 16
        neg_inf = jnp.full((16,), jnp.int32(-2147483648))

        def lane_scalar(v, t):
            return jnp.max(jnp.where(lane_iota == t, v, neg_inf))

        def fire(i_scalar, slot):
            lane0 = pl.multiple_of(i_scalar & ~lanes_mask, _LSLICE)
            pltpu.async_copy(
                table_hbm.at[:, pl.ds(lane0, _LSLICE)],
                ring_v.at[slot],
                sem,
            )

        def wait_slot(slot):
            pltpu.make_async_copy(
                table_hbm.at[:, pl.ds(0, _LSLICE)],
                ring_v.at[slot],
                sem,
            ).wait()

        def extract(i_scalar, j, slot):
            l = i_scalar & lanes_mask
            l_vec = jnp.full((16,), l, jnp.int32)
            j_vec = jnp.full((16,), j, jnp.int32)
            buf = ring_v.at[slot]
            v0 = plsc.load_gather(buf, [d_lo, l_vec])
            v1 = plsc.load_gather(buf, [d_hi, l_vec])
            plsc.store_scatter(stage_v, [d_lo, j_vec], v0)
            plsc.store_scatter(stage_v, [d_hi, j_vec], v1)

        n_grp = bpw // _NBUF

        v0g = idx_v[pl.ds(0, _NBUF)]
        for t in range(_NBUF):
            fire(lane_scalar(v0g, t), t)

        def body(g, v_cur):
            v_next = idx_v[pl.ds((g + 1) * _NBUF, _NBUF)]
            for t in range(_NBUF):
                j = g * _NBUF + t
                wait_slot(t)
                extract(lane_scalar(v_cur, t), j, t)
                fire(lane_scalar(v_next, t), t)
            return v_next

        v_last = lax.fori_loop(0, n_grp - 1, body, v0g)
        for t in range(_NBUF):
            j = (n_grp - 1) * _NBUF + t
            wait_slot(t)
            extract(lane_scalar(v_last, t), j, t)

        pltpu.sync_copy(stage_v, out_hbm.at[:, pl.ds(base, bpw)])

    return k(table_t, idx)


def _dot_t(a, b):
    return lax.dot_general(a, b, dimension_numbers=(((0,), (0,)), ((), ())),
                           preferred_element_type=jnp.float32)


def _tc_body(g_ref, s_ref, p_ref, W1_ref, b1_ref, W2_ref, b2_ref,
             Wout_ref, bout_ref, o_ref):
    s_t = jnp.log1p(s_ref[...])
    h_t = jnp.maximum(_dot_t(W1_ref[...], s_t) + b1_ref[...], 0.0)
    sv_t = _dot_t(W2_ref[...], h_t) + b2_ref[...]
    out_t = (_dot_t(Wout_ref[0:OP_DIM, :], g_ref[...])
             + _dot_t(Wout_ref[OP_DIM:OP_DIM + STATS_H, :], sv_t)
             + _dot_t(Wout_ref[OP_DIM + STATS_H:, :], p_ref[...])
             + bout_ref[...])
    o_ref[...] = out_t


def _tc_dense(g_t, stats_t, pred_t, W1, b1, W2, b2, Wout, bout):
    BLK = 8192
    col_spec = lambda d: pl.BlockSpec((d, BLK), lambda i: (0, i))
    full = lambda a: pl.BlockSpec(a.shape, lambda i: tuple(0 for _ in a.shape))
    b1_2d = b1.reshape(STATS_H, 1)
    b2_2d = b2.reshape(STATS_H, 1)
    bout_2d = bout.reshape(OUT_DIM, 1)
    return pl.pallas_call(
        _tc_body,
        grid=(B // BLK,),
        in_specs=[
            col_spec(OP_DIM),
            col_spec(STATS_IN),
            col_spec(PRED_DIM),
            full(W1), full(b1_2d), full(W2), full(b2_2d),
            full(Wout), full(bout_2d),
        ],
        out_specs=pl.BlockSpec((OUT_DIM, BLK), lambda i: (0, i)),
        out_shape=jax.ShapeDtypeStruct((OUT_DIM, B), jnp.float32),
    )(g_t, stats_t, pred_t, W1, b1_2d, W2, b2_2d, Wout, bout_2d)


def kernel(op_idx, stats, pred_feat, emb_table, W1, b1, W2, b2, Wout, bout):
    idx = op_idx.astype(jnp.int32)
    g_t = _sc_gather_t(emb_table.T, idx, B)
    out_t = _tc_dense(g_t, stats.T, pred_feat.T, W1, b1, W2, b2, Wout, bout)
    return out_t.T
